# baseline (speedup 1.0000x reference)
import functools

import jax
import jax.numpy as jnp
from jax import lax
from jax.experimental import pallas as pl
from jax.experimental.pallas import tpu as pltpu
from jax.experimental.pallas import tpu_sc as plsc

_N_WORKERS = 32
_ROWS_PER_TILE = 8
_LA = 128


def _sc_pooled_embedding(xa, xb, table, w, B, D, LA, LB):
    RT = _ROWS_PER_TILE
    LP = LA + LB
    TOK = RT * LP
    b_per_w = B // _N_WORKERS
    tiles_per_w = b_per_w // RT
    mesh = plsc.VectorSubcoreMesh(core_axis_name="c", subcore_axis_name="s")

    @functools.partial(
        pl.kernel,
        out_type=jax.ShapeDtypeStruct((B, D), jnp.float32),
        mesh=mesh,
        compiler_params=pltpu.CompilerParams(use_tc_tiling_on_sc=False),
        scratch_types=[
            pltpu.VMEM((2, RT, LA), jnp.int32),
            pltpu.VMEM((2, RT, LB), jnp.int32),
            pltpu.VMEM((2, TOK), jnp.float32),
            pltpu.VMEM((2, TOK, D), jnp.float32),
            pltpu.VMEM((b_per_w, D), jnp.float32),
            pltpu.SemaphoreType.DMA,
            pltpu.SemaphoreType.DMA,
        ],
    )
    def body(xa_hbm, xb_hbm, table_hbm, w_hbm, out_hbm,
             xa_v, xb_v, wv_v, rows_v, out_v, gsem0, gsem1):
        sid = lax.axis_index("s")
        wid = lax.axis_index("c") * (_N_WORKERS // 2) + sid
        tbase = wid * tiles_per_w

        def descs(par, sem):
            cps = []
            for r in range(RT):
                ia = xa_v.at[par, r]
                ib = xb_v.at[par, r]
                cps.append(pltpu.make_async_copy(
                    table_hbm.at[ia],
                    rows_v.at[par, pl.ds(r * LP, LA)], sem))
                cps.append(pltpu.make_async_copy(
                    table_hbm.at[ib],
                    rows_v.at[par, pl.ds(r * LP + LA, LB)], sem))
                cps.append(pltpu.make_async_copy(
                    w_hbm.at[ia],
                    wv_v.at[par, pl.ds(r * LP, LA)], sem))
                cps.append(pltpu.make_async_copy(
                    w_hbm.at[ib],
                    wv_v.at[par, pl.ds(r * LP + LA, LB)], sem))
            return cps

        def fire(t, par, sem):
            r0 = (tbase + t) * RT
            pltpu.sync_copy(xa_hbm.at[pl.ds(r0, RT)], xa_v.at[par])
            pltpu.sync_copy(xb_hbm.at[pl.ds(r0, RT)], xb_v.at[par])
            for c in descs(par, sem):
                c.start()

        def drain(par, sem):
            for c in descs(par, sem):
                c.wait()

        fire(0, 0, gsem0)

        def do_tile(t, carry):
            par = lax.rem(t, 2)

            @pl.when(t + 1 < tiles_per_w)
            def _():
                lax.cond(par == 0,
                         lambda: fire(t + 1, 1, gsem1),
                         lambda: fire(t + 1, 0, gsem0))

            lax.cond(par == 0,
                     lambda: drain(0, gsem0),
                     lambda: drain(1, gsem1))

            ii = lax.iota(jnp.int32, 16)
            for r in range(RT):
                base = r * LP
                cnt = jnp.zeros((16,), jnp.float32)
                wgs = []
                for i in range(LP // 16):
                    off = i * 16
                    if off < LA:
                        xv = xa_v[par, r, pl.ds(off, 16)]
                    else:
                        xv = xb_v[par, r, pl.ds(off - LA, 16)]
                    m = xv != 0
                    wvv = wv_v[par, pl.ds(base + off, 16)]
                    wgs.append(jnp.where(m, wvv, 0.0))
                    cnt = cnt + jnp.where(m, 1.0, 0.0)
                for sh in (8, 4, 2, 1):
                    cnt = cnt + cnt.at[ii ^ sh].get(mode="promise_in_bounds")

                accs = [jnp.zeros((16,), jnp.float32) for _ in range(D // 16)]
                for g in range(LP // 16):
                    wgv = wgs[g]
                    for u in range(16):
                        l = base + g * 16 + u
                        s = wgv.at[jnp.full((16,), u, jnp.int32)].get(
                            mode="promise_in_bounds")
                        for j in range(D // 16):
                            accs[j] = accs[j] + s * rows_v[par, l,
                                                          pl.ds(j * 16, 16)]
                inv = 1.0 / (cnt + 1e-16)
                for j in range(D // 16):
                    out_v[t * RT + r, pl.ds(j * 16, 16)] = accs[j] * inv
            return carry

        lax.fori_loop(0, tiles_per_w, do_tile, 0)
        pltpu.sync_copy(out_v, out_hbm.at[pl.ds(wid * b_per_w, b_per_w)])

    return body(xa, xb, table, w)


def _tc_linear(embedded, wt, b2, B, D, OUT):
    BM = 512

    def mm_body(e_ref, w_ref, b_ref, o_ref):
        o_ref[...] = (
            jnp.dot(e_ref[...], w_ref[...], preferred_element_type=jnp.float32)
            + b_ref[...]
        )

    return pl.pallas_call(
        mm_body,
        grid=(B // BM,),
        in_specs=[
            pl.BlockSpec((BM, D), lambda i: (i, 0)),
            pl.BlockSpec((D, OUT), lambda i: (0, 0)),
            pl.BlockSpec((1, OUT), lambda i: (0, 0)),
        ],
        out_specs=pl.BlockSpec((BM, OUT), lambda i: (i, 0)),
        out_shape=jax.ShapeDtypeStruct((B, OUT), jnp.float32),
    )(embedded, wt, b2)


def kernel(x, table, w, W_out, b_out):
    B, L = x.shape
    V, D = table.shape
    OUT = W_out.shape[0]
    LA = _LA
    LB = ((L - LA + 15) // 16) * 16

    x = x.astype(jnp.int32)
    xp = jnp.pad(x, ((0, 0), (0, LA + LB - L)))
    xa = xp[:, :LA]
    xb = xp[:, LA:]

    embedded = _sc_pooled_embedding(xa, xb, table, w, B, D, LA, LB)
    out = _tc_linear(embedded, W_out.T, b_out[None, :], B, D, OUT)
    return (out, embedded)
